# Initial kernel scaffold; baseline (speedup 1.0000x reference)
import jax
import jax.numpy as jnp
from jax.experimental import pallas as pl


def kernel(x, targets, Wq0, Wk0, Wv0):
    raise NotImplementedError("write your pallas kernel here")


# speedup vs baseline: 150.6802x; 150.6802x over previous
import jax
import jax.numpy as jnp
from jax.experimental import pallas as pl
from jax.experimental.pallas import tpu as pltpu

_ETA = 0.01
_A = 1.0 + _ETA
_T = 128
_NEWTON = 6
_H = 256


def _chunk_body(x_ref, t_ref, w0_ref, q_ref, k_ref, v_ref, wt_ref):
    c = pl.program_id(1)

    @pl.when(c == 0)
    def _init():
        wt_ref[0] = w0_ref[...]

    X = x_ref[0]
    Tt = t_ref[0]
    Wt = wt_ref[0]

    f32 = jnp.float32
    P0 = jnp.dot(X, Wt, preferred_element_type=f32)
    A = jax.lax.dot_general(X, X, (((1,), (1,)), ((), ())),
                            preferred_element_type=f32)

    ri = jax.lax.broadcasted_iota(jnp.int32, (_T, _T), 0)
    ci = jax.lax.broadcasted_iota(jnp.int32, (_T, _T), 1)
    aL = jnp.where(ri > ci, A * _A, 0.0)
    N = jnp.where(ri == ci, 1.0, 0.0) - aL
    for _ in range(_NEWTON):
        MN = N + jnp.dot(aL, N, preferred_element_type=f32)
        N = 2.0 * N - jnp.dot(N, MN, preferred_element_type=f32)

    Tt3 = jnp.concatenate([Tt, Tt, Tt], axis=1)
    R = _A * P0 - _ETA * Tt3
    U = jnp.dot(N, R, preferred_element_type=f32)
    pred = (U + _ETA * Tt3) * (1.0 / _A)

    q_ref[0] = pred[:, :_H]
    k_ref[0] = pred[:, _H:2 * _H]
    v_ref[0] = pred[:, 2 * _H:]
    wt_ref[0] = Wt - jax.lax.dot_general(X, U, (((0,), (0,)), ((), ())),
                                         preferred_element_type=f32)


def kernel(x, targets, Wq0, Wk0, Wv0):
    B, S, D = x.shape
    H = Wq0.shape[0]
    n_chunks = S // _T
    Wt0 = jnp.concatenate([Wq0, Wk0, Wv0], axis=0).T

    grid = (B, n_chunks)
    out_shapes = (
        jax.ShapeDtypeStruct((B, S, H), x.dtype),
        jax.ShapeDtypeStruct((B, S, H), x.dtype),
        jax.ShapeDtypeStruct((B, S, H), x.dtype),
        jax.ShapeDtypeStruct((B, D, 3 * H), x.dtype),
    )
    qkv_spec = pl.BlockSpec((1, _T, H), lambda b, c: (b, c, 0))
    q, k, v, WtF = pl.pallas_call(
        _chunk_body,
        grid=grid,
        in_specs=[
            pl.BlockSpec((1, _T, D), lambda b, c: (b, c, 0)),
            pl.BlockSpec((1, _T, H), lambda b, c: (b, c, 0)),
            pl.BlockSpec((D, 3 * H), lambda b, c: (0, 0)),
        ],
        out_specs=(
            qkv_spec, qkv_spec, qkv_spec,
            pl.BlockSpec((1, D, 3 * H), lambda b, c: (b, 0, 0)),
        ),
        out_shape=out_shapes,
        compiler_params=pltpu.CompilerParams(
            dimension_semantics=("parallel", "arbitrary"),
        ),
    )(x, targets, Wt0)

    Wq = WtF[:, :, :H].transpose(0, 2, 1)
    Wk = WtF[:, :, H:2 * H].transpose(0, 2, 1)
    Wv = WtF[:, :, 2 * H:].transpose(0, 2, 1)
    return q, k, v, Wq, Wk, Wv


# speedup vs baseline: 216.1778x; 1.4347x over previous
import jax
import jax.numpy as jnp
from jax.experimental import pallas as pl
from jax.experimental.pallas import tpu as pltpu

_ETA = 0.01
_A = 1.0 + _ETA
_T = 128
_NEWTON = 6
_H = 256


_G = 2


def _chunk_body(x_ref, t_ref, w0_ref, q_ref, k_ref, v_ref, wt_ref, sem):
    c = pl.program_id(1)
    G = range(_G)

    @pl.when(c == 0)
    def _init():
        for g in G:
            cp = pltpu.make_async_copy(w0_ref, wt_ref.at[g], sem)
            cp.start()
            cp.wait()

    f32 = jnp.float32

    def dot(a, b):
        return jnp.dot(a, b, preferred_element_type=f32)

    X = [x_ref[g] for g in G]
    Tt = [t_ref[g] for g in G]
    Wt = [wt_ref[g] for g in G]

    P0 = [dot(X[g], Wt[g]) for g in G]
    A = [jax.lax.dot_general(X[g], X[g], (((1,), (1,)), ((), ())),
                             preferred_element_type=f32) for g in G]

    ri = jax.lax.broadcasted_iota(jnp.int32, (_T, _T), 0)
    ci = jax.lax.broadcasted_iota(jnp.int32, (_T, _T), 1)
    aL = [jnp.where(ri > ci, A[g] * _A, 0.0) for g in G]
    N = [jnp.where(ri == ci, 1.0, 0.0) - aL[g] for g in G]
    for _ in range(_NEWTON):
        MN = [N[g] + dot(aL[g], N[g]) for g in G]
        N = [2.0 * N[g] - dot(N[g], MN[g]) for g in G]

    Tt3 = [jnp.concatenate([Tt[g], Tt[g], Tt[g]], axis=1) for g in G]
    R = [_A * P0[g] - _ETA * Tt3[g] for g in G]
    U = [dot(N[g], R[g]) for g in G]
    pred = [(U[g] + _ETA * Tt3[g]) * (1.0 / _A) for g in G]

    for g in G:
        q_ref[g] = pred[g][:, :_H]
        k_ref[g] = pred[g][:, _H:2 * _H]
        v_ref[g] = pred[g][:, 2 * _H:]
    upd = [jax.lax.dot_general(X[g], U[g], (((0,), (0,)), ((), ())),
                               preferred_element_type=f32) for g in G]
    for g in G:
        wt_ref[g] = Wt[g] - upd[g]


def kernel(x, targets, Wq0, Wk0, Wv0):
    B, S, D = x.shape
    H = Wq0.shape[0]
    n_chunks = S // _T
    Wt0 = jnp.concatenate([Wq0, Wk0, Wv0], axis=0).T

    grid = (B // _G, n_chunks)
    out_shapes = (
        jax.ShapeDtypeStruct((B, S, H), x.dtype),
        jax.ShapeDtypeStruct((B, S, H), x.dtype),
        jax.ShapeDtypeStruct((B, S, H), x.dtype),
        jax.ShapeDtypeStruct((B, D, 3 * H), x.dtype),
    )
    qkv_spec = pl.BlockSpec((_G, _T, H), lambda b, c: (b, c, 0))
    q, k, v, WtF = pl.pallas_call(
        _chunk_body,
        grid=grid,
        in_specs=[
            pl.BlockSpec((_G, _T, D), lambda b, c: (b, c, 0)),
            pl.BlockSpec((_G, _T, H), lambda b, c: (b, c, 0)),
            pl.BlockSpec((D, 3 * H), lambda b, c: (0, 0)),
        ],
        out_specs=(
            qkv_spec, qkv_spec, qkv_spec,
            pl.BlockSpec((_G, D, 3 * H), lambda b, c: (b, 0, 0)),
        ),
        out_shape=out_shapes,
        scratch_shapes=[pltpu.SemaphoreType.DMA],
        compiler_params=pltpu.CompilerParams(
            dimension_semantics=("parallel", "arbitrary"),
        ),
    )(x, targets, Wt0)

    Wq = WtF[:, :, :H].transpose(0, 2, 1)
    Wk = WtF[:, :, H:2 * H].transpose(0, 2, 1)
    Wv = WtF[:, :, 2 * H:].transpose(0, 2, 1)
    return q, k, v, Wq, Wk, Wv


# speedup vs baseline: 287.6463x; 1.3306x over previous
import jax
import jax.numpy as jnp
from jax.experimental import pallas as pl
from jax.experimental.pallas import tpu as pltpu

_ETA = 0.01
_A = 1.0 + _ETA
_T = 128
_NEWTON = 6
_H = 256


_G = 4


def _chunk_body(x_ref, t_ref, w0_ref, q_ref, k_ref, v_ref, wt_ref, sem):
    c = pl.program_id(1)
    G = range(_G)

    @pl.when(c == 0)
    def _init():
        for g in G:
            cp = pltpu.make_async_copy(w0_ref, wt_ref.at[g], sem)
            cp.start()
            cp.wait()

    f32 = jnp.float32

    def dot(a, b):
        return jnp.dot(a, b, preferred_element_type=f32)

    X = [x_ref[g] for g in G]
    Tt = [t_ref[g] for g in G]
    Wt = [wt_ref[g] for g in G]

    P0 = [dot(X[g], Wt[g]) for g in G]
    A = [jax.lax.dot_general(X[g], X[g], (((1,), (1,)), ((), ())),
                             preferred_element_type=f32) for g in G]

    ri = jax.lax.broadcasted_iota(jnp.int32, (_T, _T), 0)
    ci = jax.lax.broadcasted_iota(jnp.int32, (_T, _T), 1)
    aL = [jnp.where(ri > ci, A[g] * _A, 0.0) for g in G]
    N = [jnp.where(ri == ci, 1.0, 0.0) - aL[g] for g in G]
    for _ in range(_NEWTON):
        MN = [N[g] + dot(aL[g], N[g]) for g in G]
        N = [2.0 * N[g] - dot(N[g], MN[g]) for g in G]

    Tt3 = [jnp.concatenate([Tt[g], Tt[g], Tt[g]], axis=1) for g in G]
    R = [_A * P0[g] - _ETA * Tt3[g] for g in G]
    U = [dot(N[g], R[g]) for g in G]
    pred = [(U[g] + _ETA * Tt3[g]) * (1.0 / _A) for g in G]

    for g in G:
        q_ref[g] = pred[g][:, :_H]
        k_ref[g] = pred[g][:, _H:2 * _H]
        v_ref[g] = pred[g][:, 2 * _H:]
    upd = [jax.lax.dot_general(X[g], U[g], (((0,), (0,)), ((), ())),
                               preferred_element_type=f32) for g in G]
    for g in G:
        wt_ref[g] = Wt[g] - upd[g]


def kernel(x, targets, Wq0, Wk0, Wv0):
    B, S, D = x.shape
    H = Wq0.shape[0]
    n_chunks = S // _T
    Wt0 = jnp.concatenate([Wq0, Wk0, Wv0], axis=0).T

    grid = (B // _G, n_chunks)
    out_shapes = (
        jax.ShapeDtypeStruct((B, S, H), x.dtype),
        jax.ShapeDtypeStruct((B, S, H), x.dtype),
        jax.ShapeDtypeStruct((B, S, H), x.dtype),
        jax.ShapeDtypeStruct((B, D, 3 * H), x.dtype),
    )
    qkv_spec = pl.BlockSpec((_G, _T, H), lambda b, c: (b, c, 0))
    q, k, v, WtF = pl.pallas_call(
        _chunk_body,
        grid=grid,
        in_specs=[
            pl.BlockSpec((_G, _T, D), lambda b, c: (b, c, 0)),
            pl.BlockSpec((_G, _T, H), lambda b, c: (b, c, 0)),
            pl.BlockSpec((D, 3 * H), lambda b, c: (0, 0)),
        ],
        out_specs=(
            qkv_spec, qkv_spec, qkv_spec,
            pl.BlockSpec((_G, D, 3 * H), lambda b, c: (b, 0, 0)),
        ),
        out_shape=out_shapes,
        scratch_shapes=[pltpu.SemaphoreType.DMA],
        compiler_params=pltpu.CompilerParams(
            dimension_semantics=("parallel", "arbitrary"),
        ),
    )(x, targets, Wt0)

    Wq = WtF[:, :, :H].transpose(0, 2, 1)
    Wk = WtF[:, :, H:2 * H].transpose(0, 2, 1)
    Wv = WtF[:, :, 2 * H:].transpose(0, 2, 1)
    return q, k, v, Wq, Wk, Wv


# speedup vs baseline: 343.5619x; 1.1944x over previous
import jax
import jax.numpy as jnp
from jax.experimental import pallas as pl
from jax.experimental.pallas import tpu as pltpu

_ETA = 0.01
_A = 1.0 + _ETA
_T = 128
_NEWTON = 6
_H = 256
_G = 4


def _chunk_body(x_ref, t_ref, w0_ref, q_ref, k_ref, v_ref,
                wq_ref, wk_ref, wv_ref, wt_ref, sem):
    c = pl.program_id(1)
    nc = pl.num_programs(1)
    G = range(_G)

    @pl.when(c == 0)
    def _init():
        for g in G:
            cp = pltpu.make_async_copy(w0_ref, wt_ref.at[g], sem)
            cp.start()
            cp.wait()

    f32 = jnp.float32

    def dot(a, b):
        return jnp.dot(a, b, preferred_element_type=f32)

    X = [x_ref[g] for g in G]
    Tt = [t_ref[g] for g in G]
    Wt = [wt_ref[g] for g in G]

    P0 = [dot(X[g], Wt[g]) for g in G]
    A = [jax.lax.dot_general(X[g], X[g], (((1,), (1,)), ((), ())),
                             preferred_element_type=f32) for g in G]

    ri = jax.lax.broadcasted_iota(jnp.int32, (_T, _T), 0)
    ci = jax.lax.broadcasted_iota(jnp.int32, (_T, _T), 1)
    aL = [jnp.where(ri > ci, A[g] * _A, 0.0) for g in G]
    N = [jnp.where(ri == ci, 1.0, 0.0) - aL[g] for g in G]
    for _ in range(_NEWTON):
        MN = [N[g] + dot(aL[g], N[g]) for g in G]
        N = [2.0 * N[g] - dot(N[g], MN[g]) for g in G]

    Tt3 = [jnp.concatenate([Tt[g], Tt[g], Tt[g]], axis=1) for g in G]
    R = [_A * P0[g] - _ETA * Tt3[g] for g in G]
    U = [dot(N[g], R[g]) for g in G]
    pred = [(U[g] + _ETA * Tt3[g]) * (1.0 / _A) for g in G]

    for g in G:
        q_ref[g] = pred[g][:, :_H]
        k_ref[g] = pred[g][:, _H:2 * _H]
        v_ref[g] = pred[g][:, 2 * _H:]
    upd = [jax.lax.dot_general(X[g], U[g], (((0,), (0,)), ((), ())),
                               preferred_element_type=f32) for g in G]
    for g in G:
        wt_ref[g] = Wt[g] - upd[g]

    @pl.when(c == nc - 1)
    def _emit_final():
        for g in G:
            Wf = wt_ref[g]
            wq_ref[g] = Wf[:, :_H].T
            wk_ref[g] = Wf[:, _H:2 * _H].T
            wv_ref[g] = Wf[:, 2 * _H:].T


def kernel(x, targets, Wq0, Wk0, Wv0):
    B, S, D = x.shape
    H = Wq0.shape[0]
    n_chunks = S // _T
    Wt0 = jnp.concatenate([Wq0, Wk0, Wv0], axis=0).T

    grid = (B // _G, n_chunks)
    out_shapes = (
        jax.ShapeDtypeStruct((B, S, H), x.dtype),
        jax.ShapeDtypeStruct((B, S, H), x.dtype),
        jax.ShapeDtypeStruct((B, S, H), x.dtype),
        jax.ShapeDtypeStruct((B, H, D), x.dtype),
        jax.ShapeDtypeStruct((B, H, D), x.dtype),
        jax.ShapeDtypeStruct((B, H, D), x.dtype),
    )
    qkv_spec = pl.BlockSpec((_G, _T, H), lambda b, c: (b, c, 0))
    w_spec = pl.BlockSpec((_G, H, D), lambda b, c: (b, 0, 0))
    return pl.pallas_call(
        _chunk_body,
        grid=grid,
        in_specs=[
            pl.BlockSpec((_G, _T, D), lambda b, c: (b, c, 0)),
            pl.BlockSpec((_G, _T, H), lambda b, c: (b, c, 0)),
            pl.BlockSpec((D, 3 * H), lambda b, c: (0, 0)),
        ],
        out_specs=(
            qkv_spec, qkv_spec, qkv_spec,
            w_spec, w_spec, w_spec,
        ),
        out_shape=out_shapes,
        scratch_shapes=[
            pltpu.VMEM((_G, D, 3 * H), jnp.float32),
            pltpu.SemaphoreType.DMA,
        ],
        compiler_params=pltpu.CompilerParams(
            dimension_semantics=("parallel", "arbitrary"),
        ),
    )(x, targets, Wt0)
